# Initial kernel scaffold; baseline (speedup 1.0000x reference)
import jax
import jax.numpy as jnp
from jax.experimental import pallas as pl


def kernel(query_state, knowledge_embeddings, knowledge_keys, Wq, bq, Wv, bv, Wo, bo):
    raise NotImplementedError("write your pallas kernel here")


# speedup vs baseline: 2.7170x; 2.7170x over previous
import functools
import math

import jax
import jax.numpy as jnp
from jax import lax
from jax.experimental import pallas as pl
from jax.experimental.pallas import tpu as pltpu
from jax.experimental.pallas import tpu_sc as plsc

_HIGH = lax.Precision.HIGHEST
_NSLOT = 8


def _topk_body(nsteps, chunk, qs_ref, wq_ref, bq_ref, keys_ref,
               w_ref, i_ref, q_ref, rv_ref, ri_ref):
    step = pl.program_id(0)
    B = qs_ref.shape[0]
    D = qs_ref.shape[1]

    @pl.when(step == 0)
    def _init():
        q_ref[...] = (
            jnp.dot(qs_ref[...], wq_ref[...],
                    preferred_element_type=jnp.float32) + bq_ref[...])
        rv_ref[...] = jnp.full((B, _NSLOT), -jnp.inf, jnp.float32)
        ri_ref[...] = jnp.zeros((B, _NSLOT), jnp.int32)

    neg = jnp.float32(-jnp.inf)
    bigi = jnp.int32(2 ** 30)

    s = lax.dot_general(q_ref[...], keys_ref[...], (((1,), (1,)), ((), ())),
                        preferred_element_type=jnp.float32)
    iota = lax.broadcasted_iota(jnp.int32, (B, chunk), 1) + step * chunk

    def amax(sv):
        m = jnp.max(sv, axis=1, keepdims=True)
        i = jnp.min(jnp.where(sv == m, iota, bigi), axis=1, keepdims=True)
        return m, i

    m1, i1 = amax(s)
    s = jnp.where(iota == i1, neg, s)
    m2, i2 = amax(s)
    s = jnp.where(iota == i2, neg, s)
    m3, _ = amax(s)
    i3 = jnp.min(jnp.where(s == m3, iota, bigi), axis=1, keepdims=True)

    pad_v = jnp.full((B, 2), neg, jnp.float32)
    pad_i = jnp.zeros((B, 2), jnp.int32)
    cv = jnp.concatenate([rv_ref[:, 0:3], m1, m2, m3, pad_v], axis=1)
    ci = jnp.concatenate([ri_ref[:, 0:3], i1, i2, i3, pad_i], axis=1)
    pos = lax.broadcasted_iota(jnp.int32, (B, _NSLOT), 1)

    def amax8(sv):
        m = jnp.max(sv, axis=1, keepdims=True)
        p = jnp.min(jnp.where(sv == m, pos, bigi), axis=1, keepdims=True)
        g = jnp.min(jnp.where(pos == p, ci, bigi), axis=1, keepdims=True)
        return m, p, g

    v1, p1, g1 = amax8(cv)
    cv = jnp.where(pos == p1, neg, cv)
    v2, p2, g2 = amax8(cv)
    cv = jnp.where(pos == p2, neg, cv)
    v3, _, g3 = amax8(cv)

    pad_v5 = jnp.full((B, _NSLOT - 3), neg, jnp.float32)
    pad_i5 = jnp.zeros((B, _NSLOT - 3), jnp.int32)
    new_rv = jnp.concatenate([v1, v2, v3, pad_v5], axis=1)
    new_ri = jnp.concatenate([g1, g2, g3, pad_i5], axis=1)
    rv_ref[...] = new_rv
    ri_ref[...] = new_ri

    @pl.when(step == nsteps - 1)
    def _finish():
        scale = jnp.float32(1.0 / math.sqrt(D))
        e = jnp.exp((new_rv - v1) * scale)
        w_ref[...] = e / jnp.sum(e, axis=1, keepdims=True)
        i_ref[...] = new_ri


def _score_topk(query_state, knowledge_keys, Wq, bq, chunk):
    B, D = query_state.shape
    K = knowledge_keys.shape[0]
    n = K // chunk
    return pl.pallas_call(
        functools.partial(_topk_body, n, chunk),
        grid=(n,),
        in_specs=[
            pl.BlockSpec((B, D), lambda i: (0, 0)),
            pl.BlockSpec((D, D), lambda i: (0, 0)),
            pl.BlockSpec((1, D), lambda i: (0, 0)),
            pl.BlockSpec((chunk, D), lambda i: (i, 0)),
        ],
        out_specs=[
            pl.BlockSpec((B, _NSLOT), lambda i: (0, 0)),
            pl.BlockSpec((B, _NSLOT), lambda i: (0, 0)),
        ],
        out_shape=[
            jax.ShapeDtypeStruct((B, _NSLOT), jnp.float32),
            jax.ShapeDtypeStruct((B, _NSLOT), jnp.int32),
        ],
        scratch_shapes=[
            pltpu.VMEM((B, D), jnp.float32),
            pltpu.VMEM((B, _NSLOT), jnp.float32),
            pltpu.VMEM((B, _NSLOT), jnp.int32),
        ],
    )(query_state, Wq, bq.reshape(1, D), knowledge_keys)


def _sc_gather(embeddings, idx_t, top_k):
    K, D = embeddings.shape
    B = idx_t.shape[1]
    info = plsc.get_sparse_core_info()
    nc, ns = info.num_cores, info.num_subcores
    nw = nc * ns
    bw = B // nw
    mesh = plsc.VectorSubcoreMesh(core_axis_name="c", subcore_axis_name="s")

    @functools.partial(
        pl.kernel, mesh=mesh,
        out_type=jax.ShapeDtypeStruct((top_k * B, D), jnp.float32),
        scratch_types=[
            pltpu.VMEM((bw,), jnp.int32),
            pltpu.VMEM((bw, D), jnp.float32),
            pltpu.SemaphoreType.DMA,
        ],
    )
    def gather_k(idx_hbm, emb_hbm, out_hbm, flat_vm, rows_vm, sem):
        wid = lax.axis_index("s") * nc + lax.axis_index("c")
        base = wid * bw
        for j in range(top_k):
            pltpu.sync_copy(idx_hbm.at[j, pl.ds(base, bw)], flat_vm)
            pltpu.async_copy(emb_hbm.at[flat_vm], rows_vm, sem).wait()
            pltpu.sync_copy(rows_vm, out_hbm.at[pl.ds(j * B + base, bw)])

    return gather_k(idx_t, embeddings)


def _combine_body(top_k, n_tok, g_ref, w_ref, wv_ref, bv_ref, wo_ref, bo_ref,
                  out_ref):
    D = wv_ref.shape[0]
    w = w_ref[...]
    g = g_ref[...]
    r = g[0] * w[:, 0:1]
    for j in range(1, top_k):
        r = r + g[j] * w[:, j:j + 1]
    v = (jnp.dot(r, wv_ref[...], precision=_HIGH,
                 preferred_element_type=jnp.float32) + bv_ref[...])
    t = (jnp.dot(v, wo_ref[...], precision=_HIGH,
                 preferred_element_type=jnp.float32) + bo_ref[...])
    for m in range(n_tok):
        out_ref[m] = t[:, m * D:(m + 1) * D]


def _combine(gathered, weights, Wv, bv, Wo, bo, top_k):
    B = weights.shape[0]
    D = Wv.shape[0]
    OD = Wo.shape[1]
    n_tok = OD // D
    return pl.pallas_call(
        functools.partial(_combine_body, top_k, n_tok),
        in_specs=[
            pl.BlockSpec((top_k, B, D), lambda: (0, 0, 0)),
            pl.BlockSpec((B, _NSLOT), lambda: (0, 0)),
            pl.BlockSpec((D, D), lambda: (0, 0)),
            pl.BlockSpec((1, D), lambda: (0, 0)),
            pl.BlockSpec((D, OD), lambda: (0, 0)),
            pl.BlockSpec((1, OD), lambda: (0, 0)),
        ],
        out_specs=pl.BlockSpec((n_tok, B, D), lambda: (0, 0, 0)),
        out_shape=jax.ShapeDtypeStruct((n_tok, B, D), jnp.float32),
    )(gathered, weights, Wv, bv.reshape(1, D), Wo, bo.reshape(1, OD))


def _pick_chunk(K):
    best = 8
    for c in range(8, 4097, 8):
        if K % c == 0:
            best = c
    return best


def kernel(query_state, knowledge_embeddings, knowledge_keys,
           Wq, bq, Wv, bv, Wo, bo):
    if query_state.ndim == 1:
        query_state = query_state[None, :]
    B, D = query_state.shape
    K = knowledge_keys.shape[0]
    top_k = min(3, K)
    chunk = _pick_chunk(K)

    weights, idx = _score_topk(query_state, knowledge_keys, Wq, bq, chunk)
    idx_t = idx[:, :top_k].T
    gathered = _sc_gather(knowledge_embeddings, idx_t, top_k)
    gathered = gathered.reshape(top_k, B, D)
    return _combine(gathered, weights, Wv, bv, Wo, bo, top_k)


# speedup vs baseline: 4.4521x; 1.6386x over previous
import functools
import math

import jax
import jax.numpy as jnp
from jax import lax
from jax.experimental import pallas as pl
from jax.experimental.pallas import tpu as pltpu
from jax.experimental.pallas import tpu_sc as plsc

_HIGH = lax.Precision.HIGHEST
_NSLOT = 8
_LANES = 128
_BT = 128


def _topk_body(nsteps, chunk, qs_ref, wq_ref, bq_ref, keys_ref,
               w_ref, i_ref, q_ref, a1_ref, a2_ref, a3_ref,
               b1_ref, b2_ref, b3_ref):
    step = pl.program_id(0)
    B = qs_ref.shape[0]
    D = qs_ref.shape[1]
    neg = jnp.float32(-jnp.inf)
    bigi = jnp.int32(2 ** 30)

    @pl.when(step == 0)
    def _init():
        q_ref[...] = (
            jnp.dot(qs_ref[...], wq_ref[...],
                    preferred_element_type=jnp.float32) + bq_ref[...])
        for r in (a1_ref, a2_ref, a3_ref):
            r[...] = jnp.full((B, _LANES), neg, jnp.float32)
        for r in (b1_ref, b2_ref, b3_ref):
            r[...] = jnp.zeros((B, _LANES), jnp.int32)

    s = lax.dot_general(q_ref[...], keys_ref[...], (((1,), (1,)), ((), ())),
                        preferred_element_type=jnp.float32)

    ngrp = chunk // _LANES
    tail = chunk - ngrp * _LANES
    for t0 in range(0, B, _BT):
        ts = pl.ds(t0, _BT)
        a1 = a1_ref[ts, :]
        a2 = a2_ref[ts, :]
        a3 = a3_ref[ts, :]
        b1 = b1_ref[ts, :]
        b2 = b2_ref[ts, :]
        b3 = b3_ref[ts, :]
        for g in range(ngrp + (1 if tail else 0)):
            if g < ngrp:
                v = s[t0:t0 + _BT, g * _LANES:(g + 1) * _LANES]
            else:
                v = jnp.concatenate(
                    [s[t0:t0 + _BT, ngrp * _LANES:],
                     jnp.full((_BT, _LANES - tail), neg, jnp.float32)], axis=1)
            base = jnp.int32(step * chunk + g * _LANES)
            c1 = v > a1
            c2 = v > a2
            c3 = v > a3
            na3 = jnp.maximum(a3, jnp.minimum(a2, v))
            na2 = jnp.maximum(a2, jnp.minimum(a1, v))
            na1 = jnp.maximum(a1, v)
            nb3 = jnp.where(c2, b2, jnp.where(c3, base, b3))
            nb2 = jnp.where(c1, b1, jnp.where(c2, base, b2))
            nb1 = jnp.where(c1, base, b1)
            a1, a2, a3, b1, b2, b3 = na1, na2, na3, nb1, nb2, nb3
        a1_ref[ts, :] = a1
        a2_ref[ts, :] = a2
        a3_ref[ts, :] = a3
        b1_ref[ts, :] = b1
        b2_ref[ts, :] = b2
        b3_ref[ts, :] = b3

    @pl.when(step == nsteps - 1)
    def _finish():
        lane = lax.broadcasted_iota(jnp.int32, (B, _LANES), 1)
        a1 = a1_ref[...]
        a2 = a2_ref[...]
        a3 = a3_ref[...]
        i1 = b1_ref[...] + lane
        i2 = b2_ref[...] + lane
        i3 = b3_ref[...] + lane

        def extract(a1, a2, a3, i1, i2, i3):
            m = jnp.max(a1, axis=1, keepdims=True)
            gi = jnp.min(jnp.where(a1 == m, i1, bigi), axis=1, keepdims=True)
            pop = i1 == gi
            na1 = jnp.where(pop, a2, a1)
            ni1 = jnp.where(pop, i2, i1)
            na2 = jnp.where(pop, a3, a2)
            ni2 = jnp.where(pop, i3, i2)
            na3 = jnp.where(pop, neg, a3)
            return m, gi, na1, na2, na3, ni1, ni2, i3

        m1, g1, a1, a2, a3, i1, i2, i3 = extract(a1, a2, a3, i1, i2, i3)
        m2, g2, a1, a2, a3, i1, i2, i3 = extract(a1, a2, a3, i1, i2, i3)
        m3, g3, _, _, _, _, _, _ = extract(a1, a2, a3, i1, i2, i3)

        scale = jnp.float32(1.0 / math.sqrt(D))
        e1 = jnp.ones((B, 1), jnp.float32)
        e2 = jnp.exp((m2 - m1) * scale)
        e3 = jnp.exp((m3 - m1) * scale)
        den = e1 + e2 + e3
        zf = jnp.zeros((B, _NSLOT - 3), jnp.float32)
        zi = jnp.zeros((B, _NSLOT - 3), jnp.int32)
        w_ref[...] = jnp.concatenate([e1 / den, e2 / den, e3 / den, zf],
                                     axis=1)
        i_ref[...] = jnp.concatenate([g1, g2, g3, zi], axis=1)


def _score_topk(query_state, knowledge_keys, Wq, bq, chunk):
    B, D = query_state.shape
    K = knowledge_keys.shape[0]
    n = K // chunk
    return pl.pallas_call(
        functools.partial(_topk_body, n, chunk),
        grid=(n,),
        in_specs=[
            pl.BlockSpec((B, D), lambda i: (0, 0)),
            pl.BlockSpec((D, D), lambda i: (0, 0)),
            pl.BlockSpec((1, D), lambda i: (0, 0)),
            pl.BlockSpec((chunk, D), lambda i: (i, 0)),
        ],
        out_specs=[
            pl.BlockSpec((B, _NSLOT), lambda i: (0, 0)),
            pl.BlockSpec((B, _NSLOT), lambda i: (0, 0)),
        ],
        out_shape=[
            jax.ShapeDtypeStruct((B, _NSLOT), jnp.float32),
            jax.ShapeDtypeStruct((B, _NSLOT), jnp.int32),
        ],
        scratch_shapes=[
            pltpu.VMEM((B, D), jnp.float32),
            pltpu.VMEM((B, _LANES), jnp.float32),
            pltpu.VMEM((B, _LANES), jnp.float32),
            pltpu.VMEM((B, _LANES), jnp.float32),
            pltpu.VMEM((B, _LANES), jnp.int32),
            pltpu.VMEM((B, _LANES), jnp.int32),
            pltpu.VMEM((B, _LANES), jnp.int32),
        ],
    )(query_state, Wq, bq.reshape(1, D), knowledge_keys)


def _sc_gather(embeddings, idx_t, top_k):
    K, D = embeddings.shape
    B = idx_t.shape[1]
    info = plsc.get_sparse_core_info()
    nc, ns = info.num_cores, info.num_subcores
    nw = nc * ns
    bw = B // nw
    mesh = plsc.VectorSubcoreMesh(core_axis_name="c", subcore_axis_name="s")

    @functools.partial(
        pl.kernel, mesh=mesh,
        out_type=jax.ShapeDtypeStruct((top_k * B, D), jnp.float32),
        scratch_types=[
            pltpu.VMEM((bw,), jnp.int32),
            pltpu.VMEM((bw, D), jnp.float32),
            pltpu.SemaphoreType.DMA,
        ],
    )
    def gather_k(idx_hbm, emb_hbm, out_hbm, flat_vm, rows_vm, sem):
        wid = lax.axis_index("s") * nc + lax.axis_index("c")
        base = wid * bw
        for j in range(top_k):
            pltpu.sync_copy(idx_hbm.at[j, pl.ds(base, bw)], flat_vm)
            pltpu.async_copy(emb_hbm.at[flat_vm], rows_vm, sem).wait()
            pltpu.sync_copy(rows_vm, out_hbm.at[pl.ds(j * B + base, bw)])

    return gather_k(idx_t, embeddings)


def _combine_body(top_k, n_tok, g_ref, w_ref, wv_ref, bv_ref, wo_ref, bo_ref,
                  out_ref):
    D = wv_ref.shape[0]
    w = w_ref[...]
    g = g_ref[...]
    r = g[0] * w[:, 0:1]
    for j in range(1, top_k):
        r = r + g[j] * w[:, j:j + 1]
    v = (jnp.dot(r, wv_ref[...], precision=_HIGH,
                 preferred_element_type=jnp.float32) + bv_ref[...])
    t = (jnp.dot(v, wo_ref[...], precision=_HIGH,
                 preferred_element_type=jnp.float32) + bo_ref[...])
    for m in range(n_tok):
        out_ref[m] = t[:, m * D:(m + 1) * D]


def _combine(gathered, weights, Wv, bv, Wo, bo, top_k):
    B = weights.shape[0]
    D = Wv.shape[0]
    OD = Wo.shape[1]
    n_tok = OD // D
    return pl.pallas_call(
        functools.partial(_combine_body, top_k, n_tok),
        in_specs=[
            pl.BlockSpec((top_k, B, D), lambda: (0, 0, 0)),
            pl.BlockSpec((B, _NSLOT), lambda: (0, 0)),
            pl.BlockSpec((D, D), lambda: (0, 0)),
            pl.BlockSpec((1, D), lambda: (0, 0)),
            pl.BlockSpec((D, OD), lambda: (0, 0)),
            pl.BlockSpec((1, OD), lambda: (0, 0)),
        ],
        out_specs=pl.BlockSpec((n_tok, B, D), lambda: (0, 0, 0)),
        out_shape=jax.ShapeDtypeStruct((n_tok, B, D), jnp.float32),
    )(gathered, weights, Wv, bv.reshape(1, D), Wo, bo.reshape(1, OD))


def _pick_chunk(K):
    best = 8
    for c in range(8, 4097, 8):
        if K % c == 0:
            best = c
    return best


def kernel(query_state, knowledge_embeddings, knowledge_keys,
           Wq, bq, Wv, bv, Wo, bo):
    if query_state.ndim == 1:
        query_state = query_state[None, :]
    B, D = query_state.shape
    K = knowledge_keys.shape[0]
    top_k = min(3, K)
    chunk = _pick_chunk(K)

    weights, idx = _score_topk(query_state, knowledge_keys, Wq, bq, chunk)
    idx_t = idx[:, :top_k].T
    gathered = _sc_gather(knowledge_embeddings, idx_t, top_k)
    gathered = gathered.reshape(top_k, B, D)
    return _combine(gathered, weights, Wv, bv, Wo, bo, top_k)


# speedup vs baseline: 4.6361x; 1.0413x over previous
import functools
import math

import jax
import jax.numpy as jnp
from jax import lax
from jax.experimental import pallas as pl
from jax.experimental.pallas import tpu as pltpu
from jax.experimental.pallas import tpu_sc as plsc

_HIGH = lax.Precision.HIGHEST
_NSLOT = 8
_LANES = 128
_BT = 128


def _topk_body(nsteps, chunk, qs_ref, wq_ref, bq_ref, keys_ref,
               w_ref, i_ref, q_ref, a1_ref, a2_ref, a3_ref,
               b1_ref, b2_ref, b3_ref):
    step = pl.program_id(0)
    B = qs_ref.shape[0]
    D = qs_ref.shape[1]
    neg = jnp.float32(-jnp.inf)
    bigi = jnp.int32(2 ** 30)

    @pl.when(step == 0)
    def _init():
        q_ref[...] = (
            jnp.dot(qs_ref[...], wq_ref[...],
                    preferred_element_type=jnp.float32) + bq_ref[...])
        for r in (a1_ref, a2_ref, a3_ref):
            r[...] = jnp.full((B, _LANES), neg, jnp.float32)
        for r in (b1_ref, b2_ref, b3_ref):
            r[...] = jnp.zeros((B, _LANES), jnp.int32)

    s = lax.dot_general(q_ref[...], keys_ref[...], (((1,), (1,)), ((), ())),
                        preferred_element_type=jnp.float32)

    ngrp = chunk // _LANES
    tail = chunk - ngrp * _LANES
    for t0 in range(0, B, _BT):
        ts = pl.ds(t0, _BT)
        a1 = a1_ref[ts, :]
        a2 = a2_ref[ts, :]
        a3 = a3_ref[ts, :]
        b1 = b1_ref[ts, :]
        b2 = b2_ref[ts, :]
        b3 = b3_ref[ts, :]
        for g in range(ngrp + (1 if tail else 0)):
            if g < ngrp:
                v = s[t0:t0 + _BT, g * _LANES:(g + 1) * _LANES]
            else:
                v = jnp.concatenate(
                    [s[t0:t0 + _BT, ngrp * _LANES:],
                     jnp.full((_BT, _LANES - tail), neg, jnp.float32)], axis=1)
            base = jnp.int32(step * chunk + g * _LANES)
            c1 = v > a1
            c2 = v > a2
            c3 = v > a3
            na3 = jnp.maximum(a3, jnp.minimum(a2, v))
            na2 = jnp.maximum(a2, jnp.minimum(a1, v))
            na1 = jnp.maximum(a1, v)
            nb3 = jnp.where(c2, b2, jnp.where(c3, base, b3))
            nb2 = jnp.where(c1, b1, jnp.where(c2, base, b2))
            nb1 = jnp.where(c1, base, b1)
            a1, a2, a3, b1, b2, b3 = na1, na2, na3, nb1, nb2, nb3
        a1_ref[ts, :] = a1
        a2_ref[ts, :] = a2
        a3_ref[ts, :] = a3
        b1_ref[ts, :] = b1
        b2_ref[ts, :] = b2
        b3_ref[ts, :] = b3

    @pl.when(step == nsteps - 1)
    def _finish():
        lane = lax.broadcasted_iota(jnp.int32, (B, _LANES), 1)
        a1 = a1_ref[...]
        a2 = a2_ref[...]
        a3 = a3_ref[...]
        i1 = b1_ref[...] + lane
        i2 = b2_ref[...] + lane
        i3 = b3_ref[...] + lane

        def extract(a1, a2, a3, i1, i2, i3):
            m = jnp.max(a1, axis=1, keepdims=True)
            gi = jnp.min(jnp.where(a1 == m, i1, bigi), axis=1, keepdims=True)
            pop = i1 == gi
            na1 = jnp.where(pop, a2, a1)
            ni1 = jnp.where(pop, i2, i1)
            na2 = jnp.where(pop, a3, a2)
            ni2 = jnp.where(pop, i3, i2)
            na3 = jnp.where(pop, neg, a3)
            return m, gi, na1, na2, na3, ni1, ni2, i3

        m1, g1, a1, a2, a3, i1, i2, i3 = extract(a1, a2, a3, i1, i2, i3)
        m2, g2, a1, a2, a3, i1, i2, i3 = extract(a1, a2, a3, i1, i2, i3)
        m3, g3, _, _, _, _, _, _ = extract(a1, a2, a3, i1, i2, i3)

        scale = jnp.float32(1.0 / math.sqrt(D))
        e1 = jnp.ones((B, 1), jnp.float32)
        e2 = jnp.exp((m2 - m1) * scale)
        e3 = jnp.exp((m3 - m1) * scale)
        den = e1 + e2 + e3
        zf = jnp.zeros((B, _NSLOT - 3), jnp.float32)
        zi = jnp.zeros((B, _NSLOT - 3), jnp.int32)
        w_ref[...] = jnp.concatenate([e1 / den, e2 / den, e3 / den, zf],
                                     axis=1)
        i_ref[...] = jnp.concatenate([g1, g2, g3, zi], axis=1)


def _topk_fast_body(nsteps, chunk, qs_ref, wq_ref, bq_ref, keys_ref,
                    w_ref, i_ref, f_ref, q_ref, a1_ref, a2_ref, a3_ref,
                    b1_ref, b2_ref):
    step = pl.program_id(0)
    B = qs_ref.shape[0]
    D = qs_ref.shape[1]
    neg = jnp.float32(-jnp.inf)
    bigi = jnp.int32(2 ** 30)

    @pl.when(step == 0)
    def _init():
        q_ref[...] = (
            jnp.dot(qs_ref[...], wq_ref[...],
                    preferred_element_type=jnp.float32) + bq_ref[...])
        for r in (a1_ref, a2_ref, a3_ref):
            r[...] = jnp.full((B, _LANES), neg, jnp.float32)
        for r in (b1_ref, b2_ref):
            r[...] = jnp.zeros((B, _LANES), jnp.int32)

    s = lax.dot_general(q_ref[...], keys_ref[...], (((1,), (1,)), ((), ())),
                        preferred_element_type=jnp.float32)

    ngrp = chunk // _LANES
    tail = chunk - ngrp * _LANES
    for t0 in range(0, B, _BT):
        ts = pl.ds(t0, _BT)
        a1 = a1_ref[ts, :]
        a2 = a2_ref[ts, :]
        a3 = a3_ref[ts, :]
        b1 = b1_ref[ts, :]
        b2 = b2_ref[ts, :]
        for g in range(ngrp + (1 if tail else 0)):
            if g < ngrp:
                v = s[t0:t0 + _BT, g * _LANES:(g + 1) * _LANES]
            else:
                v = jnp.concatenate(
                    [s[t0:t0 + _BT, ngrp * _LANES:],
                     jnp.full((_BT, _LANES - tail), neg, jnp.float32)], axis=1)
            base = jnp.int32(step * chunk + g * _LANES)
            c1 = v > a1
            c2 = v > a2
            na3 = jnp.maximum(a3, jnp.minimum(a2, v))
            na2 = jnp.maximum(a2, jnp.minimum(a1, v))
            na1 = jnp.maximum(a1, v)
            nb2 = jnp.where(c1, b1, jnp.where(c2, base, b2))
            nb1 = jnp.where(c1, base, b1)
            a1, a2, a3, b1, b2 = na1, na2, na3, nb1, nb2
        a1_ref[ts, :] = a1
        a2_ref[ts, :] = a2
        a3_ref[ts, :] = a3
        b1_ref[ts, :] = b1
        b2_ref[ts, :] = b2

    @pl.when(step == nsteps - 1)
    def _finish():
        lane = lax.broadcasted_iota(jnp.int32, (B, _LANES), 1)
        a1 = a1_ref[...]
        a2 = a2_ref[...]
        a3 = a3_ref[...]
        i1 = b1_ref[...] + lane
        i2 = b2_ref[...] + lane
        i3 = jnp.full((B, _LANES), bigi, jnp.int32)

        def extract(a1, a2, a3, i1, i2, i3):
            m = jnp.max(a1, axis=1, keepdims=True)
            gi = jnp.min(jnp.where(a1 == m, i1, bigi), axis=1, keepdims=True)
            bad = jnp.any((a1 == m) & (i1 >= bigi), axis=1, keepdims=True)
            pop = i1 == gi
            na1 = jnp.where(pop, a2, a1)
            ni1 = jnp.where(pop, i2, i1)
            na2 = jnp.where(pop, a3, a2)
            ni2 = jnp.where(pop, i3, i2)
            na3 = jnp.where(pop, neg, a3)
            return m, gi, bad, na1, na2, na3, ni1, ni2, i3

        m1, g1, _, a1, a2, a3, i1, i2, i3 = extract(a1, a2, a3, i1, i2, i3)
        m2, g2, bad2, a1, a2, a3, i1, i2, i3 = extract(a1, a2, a3, i1, i2, i3)
        m3, g3, bad3, _, _, _, _, _, _ = extract(a1, a2, a3, i1, i2, i3)
        flag = jnp.max((bad2 | bad3).astype(jnp.int32))
        f_ref[...] = jnp.full((1, _NSLOT), flag, jnp.int32)

        scale = jnp.float32(1.0 / math.sqrt(D))
        e1 = jnp.ones((B, 1), jnp.float32)
        e2 = jnp.exp((m2 - m1) * scale)
        e3 = jnp.exp((m3 - m1) * scale)
        den = e1 + e2 + e3
        zf = jnp.zeros((B, _NSLOT - 3), jnp.float32)
        zi = jnp.zeros((B, _NSLOT - 3), jnp.int32)
        w_ref[...] = jnp.concatenate([e1 / den, e2 / den, e3 / den, zf],
                                     axis=1)
        i_ref[...] = jnp.concatenate([g1, g2, g3, zi], axis=1)


def _score_topk_fast(query_state, knowledge_keys, Wq, bq, chunk):
    B, D = query_state.shape
    K = knowledge_keys.shape[0]
    n = K // chunk
    return pl.pallas_call(
        functools.partial(_topk_fast_body, n, chunk),
        grid=(n,),
        in_specs=[
            pl.BlockSpec((B, D), lambda i: (0, 0)),
            pl.BlockSpec((D, D), lambda i: (0, 0)),
            pl.BlockSpec((1, D), lambda i: (0, 0)),
            pl.BlockSpec((chunk, D), lambda i: (i, 0)),
        ],
        out_specs=[
            pl.BlockSpec((B, _NSLOT), lambda i: (0, 0)),
            pl.BlockSpec((B, _NSLOT), lambda i: (0, 0)),
            pl.BlockSpec((1, _NSLOT), lambda i: (0, 0)),
        ],
        out_shape=[
            jax.ShapeDtypeStruct((B, _NSLOT), jnp.float32),
            jax.ShapeDtypeStruct((B, _NSLOT), jnp.int32),
            jax.ShapeDtypeStruct((1, _NSLOT), jnp.int32),
        ],
        scratch_shapes=[
            pltpu.VMEM((B, D), jnp.float32),
            pltpu.VMEM((B, _LANES), jnp.float32),
            pltpu.VMEM((B, _LANES), jnp.float32),
            pltpu.VMEM((B, _LANES), jnp.float32),
            pltpu.VMEM((B, _LANES), jnp.int32),
            pltpu.VMEM((B, _LANES), jnp.int32),
        ],
    )(query_state, Wq, bq.reshape(1, D), knowledge_keys)


def _score_topk(query_state, knowledge_keys, Wq, bq, chunk):
    B, D = query_state.shape
    K = knowledge_keys.shape[0]
    n = K // chunk
    return pl.pallas_call(
        functools.partial(_topk_body, n, chunk),
        grid=(n,),
        in_specs=[
            pl.BlockSpec((B, D), lambda i: (0, 0)),
            pl.BlockSpec((D, D), lambda i: (0, 0)),
            pl.BlockSpec((1, D), lambda i: (0, 0)),
            pl.BlockSpec((chunk, D), lambda i: (i, 0)),
        ],
        out_specs=[
            pl.BlockSpec((B, _NSLOT), lambda i: (0, 0)),
            pl.BlockSpec((B, _NSLOT), lambda i: (0, 0)),
        ],
        out_shape=[
            jax.ShapeDtypeStruct((B, _NSLOT), jnp.float32),
            jax.ShapeDtypeStruct((B, _NSLOT), jnp.int32),
        ],
        scratch_shapes=[
            pltpu.VMEM((B, D), jnp.float32),
            pltpu.VMEM((B, _LANES), jnp.float32),
            pltpu.VMEM((B, _LANES), jnp.float32),
            pltpu.VMEM((B, _LANES), jnp.float32),
            pltpu.VMEM((B, _LANES), jnp.int32),
            pltpu.VMEM((B, _LANES), jnp.int32),
            pltpu.VMEM((B, _LANES), jnp.int32),
        ],
    )(query_state, Wq, bq.reshape(1, D), knowledge_keys)


def _sc_gather(embeddings, idx_t, top_k):
    K, D = embeddings.shape
    B = idx_t.shape[1]
    info = plsc.get_sparse_core_info()
    nc, ns = info.num_cores, info.num_subcores
    nw = nc * ns
    bw = B // nw
    mesh = plsc.VectorSubcoreMesh(core_axis_name="c", subcore_axis_name="s")

    @functools.partial(
        pl.kernel, mesh=mesh,
        out_type=jax.ShapeDtypeStruct((top_k * B, D), jnp.float32),
        scratch_types=[
            pltpu.VMEM((bw,), jnp.int32),
            pltpu.VMEM((bw, D), jnp.float32),
            pltpu.SemaphoreType.DMA,
        ],
    )
    def gather_k(idx_hbm, emb_hbm, out_hbm, flat_vm, rows_vm, sem):
        wid = lax.axis_index("s") * nc + lax.axis_index("c")
        base = wid * bw
        for j in range(top_k):
            pltpu.sync_copy(idx_hbm.at[j, pl.ds(base, bw)], flat_vm)
            pltpu.async_copy(emb_hbm.at[flat_vm], rows_vm, sem).wait()
            pltpu.sync_copy(rows_vm, out_hbm.at[pl.ds(j * B + base, bw)])

    return gather_k(idx_t, embeddings)


def _combine_body(top_k, n_tok, g_ref, w_ref, wv_ref, bv_ref, wo_ref, bo_ref,
                  out_ref):
    D = wv_ref.shape[0]
    w = w_ref[...]
    g = g_ref[...]
    r = g[0] * w[:, 0:1]
    for j in range(1, top_k):
        r = r + g[j] * w[:, j:j + 1]
    v = (jnp.dot(r, wv_ref[...], precision=_HIGH,
                 preferred_element_type=jnp.float32) + bv_ref[...])
    t = (jnp.dot(v, wo_ref[...], precision=_HIGH,
                 preferred_element_type=jnp.float32) + bo_ref[...])
    for m in range(n_tok):
        out_ref[m] = t[:, m * D:(m + 1) * D]


def _combine(gathered, weights, Wv, bv, Wo, bo, top_k):
    B = weights.shape[0]
    D = Wv.shape[0]
    OD = Wo.shape[1]
    n_tok = OD // D
    return pl.pallas_call(
        functools.partial(_combine_body, top_k, n_tok),
        in_specs=[
            pl.BlockSpec((top_k, B, D), lambda: (0, 0, 0)),
            pl.BlockSpec((B, _NSLOT), lambda: (0, 0)),
            pl.BlockSpec((D, D), lambda: (0, 0)),
            pl.BlockSpec((1, D), lambda: (0, 0)),
            pl.BlockSpec((D, OD), lambda: (0, 0)),
            pl.BlockSpec((1, OD), lambda: (0, 0)),
        ],
        out_specs=pl.BlockSpec((n_tok, B, D), lambda: (0, 0, 0)),
        out_shape=jax.ShapeDtypeStruct((n_tok, B, D), jnp.float32),
    )(gathered, weights, Wv, bv.reshape(1, D), Wo, bo.reshape(1, OD))


def _pick_chunk(K, cap=4096):
    best = 8
    for c in range(8, cap + 1, 8):
        if K % c == 0:
            best = c
    return best


def kernel(query_state, knowledge_embeddings, knowledge_keys,
           Wq, bq, Wv, bv, Wo, bo):
    if query_state.ndim == 1:
        query_state = query_state[None, :]
    B, D = query_state.shape
    K = knowledge_keys.shape[0]
    top_k = min(3, K)
    chunk = _pick_chunk(K)

    weights_f, idx_f, flag = _score_topk_fast(
        query_state, knowledge_keys, Wq, bq, chunk)
    weights, idx = lax.cond(
        flag[0, 0] > 0,
        lambda: tuple(_score_topk(query_state, knowledge_keys, Wq, bq, chunk)),
        lambda: (weights_f, idx_f))
    idx_t = idx[:, :top_k].T
    gathered = _sc_gather(knowledge_embeddings, idx_t, top_k)
    gathered = gathered.reshape(top_k, B, D)
    return _combine(gathered, weights, Wv, bv, Wo, bo, top_k)
